# baseline (speedup 1.0000x reference)
import functools

import jax
import jax.numpy as jnp
from jax import lax
from jax.experimental import pallas as pl
from jax.experimental.pallas import tpu as pltpu
from jax.experimental.pallas import tpu_sc as plsc

BUCKET_NUMBER = 1000
MIN_VALUE = 0.0
MAX_VALUE = 1.0
STEP = (MAX_VALUE - MIN_VALUE) / BUCKET_NUMBER
EMBED_DIM = 64

LANES = 16
BC = 128
LAT = 16


def _make_kernel(BATCH, H, D):
    info = plsc.get_sparse_core_info()
    NC, NS = info.num_cores, info.num_subcores
    NW = NC * NS
    assert BATCH % (NW * BC) == 0 and H % 2 == 0 and D % LAT == 0
    rows_per_w = BATCH // NW
    n_blocks = rows_per_w // BC
    TSTRIDE = D + 1
    TSZ = BUCKET_NUMBER * TSTRIDE

    mesh = plsc.VectorSubcoreMesh(core_axis_name="c", subcore_axis_name="s")

    @functools.partial(
        pl.kernel,
        out_type=jax.ShapeDtypeStruct((H, D, BATCH), jnp.float32),
        mesh=mesh,
        scratch_types=[
            pltpu.VMEM((TSZ,), jnp.float32),
            pltpu.VMEM((BC * H,), jnp.float32),
            pltpu.VMEM((H, BC), jnp.int32),
            pltpu.VMEM((D, BC), jnp.float32),
            pltpu.VMEM((D, BC), jnp.float32),
            pltpu.SemaphoreType.DMA,
            pltpu.SemaphoreType.DMA,
        ],
        compiler_params=pltpu.CompilerParams(
            use_tc_tiling_on_sc=False, needs_layout_passes=False),
    )
    def k(xt_hbm, tab_hbm, out_hbm, tab_v, x_v, idx_v,
          tbuf_a, tbuf_b, ssem_a, ssem_b):
        wid = lax.axis_index("s") * NC + lax.axis_index("c")
        w0 = wid * rows_per_w
        lane = lax.iota(jnp.int32, LANES)
        xstride = lane * H
        tbufs, ssems = (tbuf_a, tbuf_b), (ssem_a, ssem_b)

        pltpu.sync_copy(tab_hbm, tab_v)

        def store_h(h, b0, kb):
            return pltpu.make_async_copy(
                tbufs[kb], out_hbm.at[h, :, pl.ds(b0, BC)], ssems[kb])

        def block_body(blk, carry):
            b0 = w0 + blk * BC
            pltpu.sync_copy(xt_hbm.at[pl.ds(b0 * H, BC * H)], x_v)

            def idx_body(h, carry2):
                for c in range(BC // LANES):
                    xi = plsc.load_gather(
                        x_v, [xstride + (c * LANES * H + h)])
                    idx = ((xi - MIN_VALUE) / STEP).astype(jnp.int32)
                    idx = jnp.minimum(jnp.maximum(idx, 0), BUCKET_NUMBER - 1)
                    idx_v[h, pl.ds(c * LANES, LANES)] = idx
                return carry2

            lax.fori_loop(0, H, idx_body, 0)

            def h_body(j, carry2):
                for kb in range(2):
                    hh = 2 * j + kb

                    @pl.when(j > 0)
                    def _(hh=hh, kb=kb):
                        store_h(hh - 2, b0, kb).wait()

                    prev = None
                    for c in range(BC // LANES):
                        iv = idx_v[hh, pl.ds(c * LANES, LANES)] * TSTRIDE
                        for db in range(0, D, LAT):
                            cur = (c, db, [
                                plsc.load_gather(tab_v, [iv + (db + q)])
                                for q in range(LAT)
                            ])
                            if prev is not None:
                                pc, pdb, pvs = prev
                                for q in range(LAT):
                                    tbufs[kb][pdb + q,
                                              pl.ds(pc * LANES, LANES)] = pvs[q]
                            prev = cur
                    pc, pdb, pvs = prev
                    for q in range(LAT):
                        tbufs[kb][pdb + q, pl.ds(pc * LANES, LANES)] = pvs[q]
                    store_h(hh, b0, kb).start()
                return carry2

            lax.fori_loop(0, H // 2, h_body, 0)

            store_h(H - 2, b0, 0).wait()
            store_h(H - 1, b0, 1).wait()
            return carry

        lax.fori_loop(0, n_blocks, block_body, 0)

    return k


def kernel(x, table):
    if x.ndim == 2 and x.shape[1] == 1:
        x = jnp.squeeze(x, axis=-1)
    BATCH, H = x.shape
    D = table.shape[1]
    xf = x.reshape(BATCH * H)
    tab = jnp.pad(table, ((0, 0), (0, 1))).reshape(BUCKET_NUMBER * (D + 1))
    out_t = _make_kernel(BATCH, H, D)(xf, tab)
    return jnp.transpose(out_t, (2, 0, 1))


# speedup vs baseline: 1.5852x; 1.5852x over previous
import functools

import jax
import jax.numpy as jnp
from jax import lax
from jax.experimental import pallas as pl
from jax.experimental.pallas import tpu as pltpu
from jax.experimental.pallas import tpu_sc as plsc

BUCKET_NUMBER = 1000
MIN_VALUE = 0.0
MAX_VALUE = 1.0
STEP = (MAX_VALUE - MIN_VALUE) / BUCKET_NUMBER
EMBED_DIM = 64

LANES = 16
BC = 128
LAT = 16
ISTRIDE = 65


def _make_kernel(BATCH, H, D):
    info = plsc.get_sparse_core_info()
    NC, NS = info.num_cores, info.num_subcores
    NW = NC * NS
    assert BATCH % (NW * BC) == 0 and H % 2 == 0 and D % LAT == 0
    rows_per_w = BATCH // NW
    n_blocks = rows_per_w // BC
    TSTRIDE = D + 1
    TSZ = BUCKET_NUMBER * TSTRIDE
    HCH = -(-H // LANES)

    mesh = plsc.VectorSubcoreMesh(core_axis_name="c", subcore_axis_name="s")

    @functools.partial(
        pl.kernel,
        out_type=jax.ShapeDtypeStruct((H, D, BATCH), jnp.float32),
        mesh=mesh,
        scratch_types=[
            pltpu.VMEM((TSZ,), jnp.float32),
            pltpu.VMEM((BC, H), jnp.float32),
            pltpu.VMEM((BC * ISTRIDE,), jnp.int32),
            pltpu.VMEM((D, BC), jnp.float32),
            pltpu.VMEM((D, BC), jnp.float32),
            pltpu.SemaphoreType.DMA,
            pltpu.SemaphoreType.DMA,
        ],
        compiler_params=pltpu.CompilerParams(
            use_tc_tiling_on_sc=True, needs_layout_passes=False),
    )
    def k(x_hbm, tab_hbm, out_hbm, tab_v, x_v, idx_v,
          tbuf_a, tbuf_b, ssem_a, ssem_b):
        wid = lax.axis_index("s") * NC + lax.axis_index("c")
        w0 = wid * rows_per_w
        lane = lax.iota(jnp.int32, LANES)
        ilane = lane * ISTRIDE
        tbufs, ssems = (tbuf_a, tbuf_b), (ssem_a, ssem_b)

        pltpu.sync_copy(tab_hbm, tab_v)

        def store_h(h, b0, kb):
            return pltpu.make_async_copy(
                tbufs[kb], out_hbm.at[h, :, pl.ds(b0, BC)], ssems[kb])

        def block_body(blk, carry):
            b0 = w0 + blk * BC
            pltpu.sync_copy(x_hbm.at[pl.ds(b0, BC), :], x_v)

            starts = list(range(0, H - LANES + 1, LANES))
            if starts[-1] != H - LANES:
                starts.append(H - LANES)

            def idx_body(b, carry2):
                for s in starts:
                    xi = x_v[b, pl.ds(s, LANES)]
                    idx = ((xi - MIN_VALUE) / STEP).astype(jnp.int32)
                    idx = jnp.minimum(jnp.maximum(idx, 0), BUCKET_NUMBER - 1)
                    idx_v[pl.ds(b * ISTRIDE + s, LANES)] = idx * TSTRIDE
                return carry2

            lax.fori_loop(0, BC, idx_body, 0)

            def h_body(j, carry2):
                for kb in range(2):
                    hh = 2 * j + kb

                    @pl.when(j > 0)
                    def _(hh=hh, kb=kb):
                        store_h(hh - 2, b0, kb).wait()

                    prev = None
                    for c in range(BC // LANES):
                        iv = plsc.load_gather(
                            idx_v, [ilane + (c * LANES * ISTRIDE + hh)])
                        for db in range(0, D, LAT):
                            cur = (c, db, [
                                plsc.load_gather(tab_v, [iv + (db + q)])
                                for q in range(LAT)
                            ])
                            if prev is not None:
                                pc, pdb, pvs = prev
                                for q in range(LAT):
                                    tbufs[kb][pdb + q,
                                              pl.ds(pc * LANES, LANES)] = pvs[q]
                            prev = cur
                    pc, pdb, pvs = prev
                    for q in range(LAT):
                        tbufs[kb][pdb + q, pl.ds(pc * LANES, LANES)] = pvs[q]
                    store_h(hh, b0, kb).start()
                return carry2

            lax.fori_loop(0, H // 2, h_body, 0)

            store_h(H - 2, b0, 0).wait()
            store_h(H - 1, b0, 1).wait()
            return carry

        lax.fori_loop(0, n_blocks, block_body, 0)

    return k


def kernel(x, table):
    if x.ndim == 2 and x.shape[1] == 1:
        x = jnp.squeeze(x, axis=-1)
    BATCH, H = x.shape
    D = table.shape[1]
    tab = jnp.pad(table, ((0, 0), (0, 1))).reshape(BUCKET_NUMBER * (D + 1))
    out_t = _make_kernel(BATCH, H, D)(x, tab)
    return jnp.transpose(out_t, (2, 0, 1))
